# baseline (speedup 1.0000x reference)
import dataclasses
import functools

import jax
import jax.numpy as jnp
from jax import lax
from jax.experimental import pallas as pl
from jax.experimental.pallas import tpu as pltpu
from jax.experimental.pallas import tpu_sc as plsc

_COMMIT = 0.25
_EPS = 1e-12

_TOK_BLOCK = 1024
_NW = 32
_CHUNK = 128


def _prep_body(cbt_ref, cb_ref, cbnt_ref, cbn_ref):
    cbt = cbt_ref[...]
    cn = jnp.sqrt(jnp.sum(cbt * cbt, axis=0, keepdims=True))
    cbnt_ref[...] = (cbt / jnp.maximum(cn, _EPS)).astype(jnp.bfloat16)
    cb = cb_ref[...]
    rn = jnp.sqrt(jnp.sum(cb * cb, axis=1, keepdims=True))
    cbn_ref[...] = cb / jnp.maximum(rn, _EPS)


def _run_prep(cbt, cb):
    d, ncb = cbt.shape
    return pl.pallas_call(
        _prep_body,
        out_shape=[
            jax.ShapeDtypeStruct((d, ncb), jnp.bfloat16),
            jax.ShapeDtypeStruct((ncb, d), jnp.float32),
        ],
    )(cbt, cb)


def _argmax_body(x_ref, cbnt_ref, idx_ref, sums_ref, acc_ref):
    i = pl.program_id(0)
    ncb = cbnt_ref.shape[1]
    tok = x_ref.shape[0]
    nsplit = 4
    part = tok // nsplit
    ntile = ncb // 128
    big = jnp.int32(2 ** 30)

    @pl.when(i == 0)
    def _init():
        acc_ref[0] = 0.0
        acc_ref[1] = 0.0

    cbnt = cbnt_ref[...]
    idx_parts = []
    sum_m = jnp.float32(0.0)
    sum_rn = jnp.float32(0.0)
    for h in range(nsplit):
        x = x_ref[pl.ds(h * part, part), :]
        rown = jnp.sqrt(jnp.sum(x * x, axis=1, keepdims=True))
        xn = x / jnp.maximum(rown, _EPS)
        sim = jnp.dot(xn.astype(jnp.bfloat16), cbnt,
                      preferred_element_type=jnp.float32)
        run_val = sim[:, 0:128]
        run_idx = jnp.zeros(run_val.shape, jnp.int32)
        for t in range(1, ntile):
            tile = sim[:, t * 128:(t + 1) * 128]
            c = tile > run_val
            run_val = jnp.where(c, tile, run_val)
            run_idx = jnp.where(c, jnp.int32(t), run_idx)
        m = jnp.max(run_val, axis=1)
        lane = lax.broadcasted_iota(jnp.int32, run_idx.shape, 1)
        cand = run_idx * 128 + lane
        idx_parts.append(jnp.min(
            jnp.where(run_val == m[:, None], cand, big), axis=1))
        sum_m = sum_m + jnp.sum(m)
        sum_rn = sum_rn + jnp.sum(rown)

    idx_ref[...] = jnp.concatenate(idx_parts).reshape(idx_ref.shape)
    acc_ref[0] += sum_m
    acc_ref[1] += sum_rn

    @pl.when(i == pl.num_programs(0) - 1)
    def _fin():
        sums_ref[0, 0] = acc_ref[0]
        sums_ref[0, 1] = acc_ref[1]


def _run_phase1(x, cbnt):
    n, d = x.shape
    ncb = cbnt.shape[1]
    steps = n // _TOK_BLOCK
    return pl.pallas_call(
        _argmax_body,
        grid=(steps,),
        in_specs=[
            pl.BlockSpec((_TOK_BLOCK, d), lambda i: (i, 0)),
            pl.BlockSpec((d, ncb), lambda i: (0, 0)),
        ],
        out_specs=[
            pl.BlockSpec((_TOK_BLOCK // _CHUNK, _CHUNK), lambda i: (i, 0)),
            pl.BlockSpec(memory_space=pltpu.SMEM),
        ],
        out_shape=[
            jax.ShapeDtypeStruct((n // _CHUNK, _CHUNK), jnp.int32),
            jax.ShapeDtypeStruct((1, 2), jnp.float32),
        ],
        scratch_shapes=[
            pltpu.SMEM((2,), jnp.float32),
        ],
    )(x, cbnt)


def _scal_body(sa_ref, sb_ref, out_ref, *, n_tok, d):
    n = jnp.float32(n_tok)
    s_m = sa_ref[0, 0] + sb_ref[0, 0]
    s_rn = sa_ref[0, 1] + sb_ref[0, 1]
    out_ref[0, 0] = (1.0 + _COMMIT) * (2.0 * n - 2.0 * s_m) / (
        n * jnp.float32(d))
    out_ref[0, 1] = s_rn / n


def _run_scal(sums_a, sums_b, n_tok, d):
    return pl.pallas_call(
        functools.partial(_scal_body, n_tok=n_tok, d=d),
        in_specs=[
            pl.BlockSpec(memory_space=pltpu.SMEM),
            pl.BlockSpec(memory_space=pltpu.SMEM),
        ],
        out_specs=pl.BlockSpec(memory_space=pltpu.SMEM),
        out_shape=jax.ShapeDtypeStruct((1, 2), jnp.float32),
    )(sums_a, sums_b)


def _run_sc(cbn, idx2d, zeros, n_half, k, d, ncb):
    per_w = n_half // _NW
    rows_half = per_w // 2
    nchunk = per_w // _CHUNK
    hist_cols = ncb // 2
    qlines = 256
    mesh = plsc.VectorSubcoreMesh(core_axis_name="c", subcore_axis_name="s")
    cp = pltpu.CompilerParams()
    if "needs_layout_passes" in pltpu.CompilerParams.__dataclass_fields__:
        cp = dataclasses.replace(cp, needs_layout_passes=False)
    if "use_tc_tiling_on_sc" in pltpu.CompilerParams.__dataclass_fields__:
        cp = dataclasses.replace(cp, use_tc_tiling_on_sc=False)

    @functools.partial(
        pl.kernel,
        mesh=mesh,
        compiler_params=cp,
        out_type=[
            jax.ShapeDtypeStruct((n_half * d // 128, 128), jnp.float32),
            jax.ShapeDtypeStruct((2, 2, 16, hist_cols), jnp.float32),
        ],
        scratch_types=[
            pltpu.VMEM((nchunk, _CHUNK), jnp.int32),
            pltpu.VMEM((rows_half, d), jnp.float32),
            pltpu.VMEM((16, hist_cols), jnp.float32),
            pltpu.VMEM_SHARED((16, hist_cols), jnp.float32),
            pltpu.SemaphoreType.DMA,
        ],
    )
    def sc_kernel(cbn_hbm, idx_hbm, zeros_hbm, q_hbm, cnt_hbm,
                  idx_v, rows_v, hist_v, sh, sem):
        cid = lax.axis_index("c")
        sid = lax.axis_index("s")
        wid = sid * 2 + cid
        pltpu.sync_copy(idx_hbm.at[pl.ds(wid * nchunk, nchunk)], idx_v)
        ones = jnp.full((16,), 1.0, jnp.float32)
        lanes = lax.iota(jnp.int32, 16)
        half_n = nchunk // 2
        sub_lines = rows_half * d // 128

        for half in range(2):
            copies = []
            for j in range(half_n):
                copies.append(pltpu.async_copy(
                    cbn_hbm.at[idx_v.at[half * half_n + j]],
                    rows_v.at[pl.ds(j * _CHUNK, _CHUNK)], sem))
            pltpu.sync_copy(zeros_hbm, hist_v)
            lo = jnp.int32(half * hist_cols)
            hi = jnp.int32((half + 1) * hist_cols)
            for j in range(nchunk):
                for kk in range(_CHUNK // 16):
                    vec = idx_v[j, pl.ds(kk * 16, 16)]
                    mask = (vec >= lo) & (vec < hi)
                    plsc.addupdate_scatter(hist_v, [lanes, vec - lo], ones,
                                           mask=mask)
            @pl.when(sid == 0)
            def _zero_shared():
                pltpu.sync_copy(zeros_hbm, sh)
            plsc.subcore_barrier()
            pltpu.sync_copy(hist_v, sh.at[lanes], add=True)
            plsc.subcore_barrier()

            @pl.when(sid == 0)
            def _spill_counts():
                pltpu.sync_copy(sh, cnt_hbm.at[half, cid])

            for cp_ in copies:
                cp_.wait()
            grp = k // 4
            ngrp = rows_half // grp
            for jj in range(ngrp):
                j = half * ngrp + jj
                pltpu.sync_copy(
                    rows_v.at[pl.ds(jj * grp, grp)],
                    q_hbm.at[pl.ds(wid * qlines, qlines),
                             pl.ds(j * d, d)])

    return sc_kernel(cbn, idx2d, zeros)


def _qt_body(qa_ref, qb_ref, scal_ref, out_ref):
    i = pl.program_id(0)
    half_steps = pl.num_programs(0) // 2
    nb, d, k = out_ref.shape
    lines = k // 4
    s = scal_ref[0, 1]

    use_a = i < half_steps
    for bb in range(nb):
        for j in range(128 // d):
            sl = (pl.ds(bb * lines, lines), pl.ds(j * d, d))
            sub = jnp.where(use_a, qa_ref[sl], qb_ref[sl])
            out_ref[bb, :, pl.ds(j * lines, lines)] = sub.T * s


def _run_qt(qa, qb, scal, b, k, d):
    nb = 8
    lines_blk = nb * (k // 4)
    nsteps = b // nb
    hs = nsteps // 2
    return pl.pallas_call(
        _qt_body,
        grid=(nsteps,),
        in_specs=[
            pl.BlockSpec((lines_blk, 128),
                         lambda i: (jnp.minimum(i, hs - 1), 0)),
            pl.BlockSpec((lines_blk, 128),
                         lambda i: (jnp.maximum(i - hs, 0), 0)),
            pl.BlockSpec(memory_space=pltpu.SMEM),
        ],
        out_specs=pl.BlockSpec((nb, d, k), lambda i: (i, 0, 0)),
        out_shape=jax.ShapeDtypeStruct((b, d, k), jnp.float32),
    )(qa, qb, scal)


def _perp_body(cnt_ref, perp_ref, *, n_tok):
    c = cnt_ref[...]
    rows = c.shape[0] // 2
    c0 = jnp.sum(c[:rows], axis=0)
    c1 = jnp.sum(c[rows:], axis=0)
    inv = jnp.float32(1.0 / n_tok)
    p0 = c0 * inv
    p1 = c1 * inv
    ent = jnp.sum(p0 * jnp.log(p0 + 1e-10)) + jnp.sum(p1 * jnp.log(p1 + 1e-10))
    perp_ref[...] = jnp.reshape(jnp.exp(-ent), (1, 1))


def _run_perp(cnt2d, n_tok):
    return pl.pallas_call(
        functools.partial(_perp_body, n_tok=n_tok),
        out_shape=jax.ShapeDtypeStruct((1, 1), jnp.float32),
    )(cnt2d)


def kernel(inputs, codebook):
    b, k, d = inputs.shape
    n_tok = b * k
    n_half = n_tok // 2
    ncb = codebook.shape[0]
    x = inputs.reshape(n_tok, d).astype(jnp.float32)
    cb = codebook.astype(jnp.float32)

    cbnt, cbn = _run_prep(cb.T, cb)
    idx_a, sums_a = _run_phase1(x[:n_half], cbnt)
    idx_b, sums_b = _run_phase1(x[n_half:], cbnt)
    zeros = jnp.zeros((16, ncb // 2), jnp.float32)
    qa, cnt_a = _run_sc(cbn, idx_a, zeros, n_half, k, d, ncb)
    qb, cnt_b = _run_sc(cbn, idx_b, zeros, n_half, k, d, ncb)
    scal = _run_scal(sums_a, sums_b, n_tok, d)
    hc = ncb // 2
    cnt2d = jnp.concatenate(
        [cnt_a[0].reshape(32, hc), cnt_b[0].reshape(32, hc),
         cnt_a[1].reshape(32, hc), cnt_b[1].reshape(32, hc)], axis=0)
    perp11 = _run_perp(cnt2d, n_tok)
    qt = _run_qt(qa, qb, scal, b, k, d)

    return (jnp.swapaxes(qt, 1, 2), scal[0, 0], perp11[0, 0])


# speedup vs baseline: 1.0616x; 1.0616x over previous
import dataclasses
import functools

import jax
import jax.numpy as jnp
from jax import lax
from jax.experimental import pallas as pl
from jax.experimental.pallas import tpu as pltpu
from jax.experimental.pallas import tpu_sc as plsc

_COMMIT = 0.25
_EPS = 1e-12

_TOK_BLOCK = 1024
_NW = 32
_CHUNK = 128


def _prep_body(cbt_ref, cbnt_ref):
    cbt = cbt_ref[...]
    cn = jnp.sqrt(jnp.sum(cbt * cbt, axis=0, keepdims=True))
    cbnt_ref[...] = (cbt / jnp.maximum(cn, _EPS)).astype(jnp.bfloat16)


def _run_prep(cbt):
    d, ncb = cbt.shape
    return pl.pallas_call(
        _prep_body,
        out_shape=jax.ShapeDtypeStruct((d, ncb), jnp.bfloat16),
    )(cbt)


def _argmax_body(x_ref, cbnt_ref, idx_ref, sums_ref, acc_ref):
    i = pl.program_id(0)
    ncb = cbnt_ref.shape[1]
    tok = x_ref.shape[0]
    nsplit = 4
    part = tok // nsplit
    ntile = ncb // 128
    big = jnp.int32(2 ** 30)

    @pl.when(i == 0)
    def _init():
        acc_ref[0] = 0.0
        acc_ref[1] = 0.0

    cbnt = cbnt_ref[...]
    idx_parts = []
    sum_m = jnp.float32(0.0)
    sum_rn = jnp.float32(0.0)
    for h in range(nsplit):
        x = x_ref[pl.ds(h * part, part), :]
        rown = jnp.sqrt(jnp.sum(x * x, axis=1, keepdims=True))
        xn = x / jnp.maximum(rown, _EPS)
        sim = jnp.dot(xn.astype(jnp.bfloat16), cbnt,
                      preferred_element_type=jnp.float32)
        run_val = sim[:, 0:128]
        run_idx = jnp.zeros(run_val.shape, jnp.int32)
        for t in range(1, ntile):
            tile = sim[:, t * 128:(t + 1) * 128]
            c = tile > run_val
            run_val = jnp.where(c, tile, run_val)
            run_idx = jnp.where(c, jnp.int32(t), run_idx)
        m = jnp.max(run_val, axis=1)
        lane = lax.broadcasted_iota(jnp.int32, run_idx.shape, 1)
        cand = run_idx * 128 + lane
        idx_parts.append(jnp.min(
            jnp.where(run_val == m[:, None], cand, big), axis=1))
        sum_m = sum_m + jnp.sum(m)
        sum_rn = sum_rn + jnp.sum(rown)

    idx_ref[...] = jnp.concatenate(idx_parts).reshape(idx_ref.shape)
    acc_ref[0] += sum_m
    acc_ref[1] += sum_rn

    @pl.when(i == pl.num_programs(0) - 1)
    def _fin():
        sums_ref[0, 0] = acc_ref[0]
        sums_ref[0, 1] = acc_ref[1]


def _run_phase1(x, cbnt):
    n, d = x.shape
    ncb = cbnt.shape[1]
    steps = n // _TOK_BLOCK
    return pl.pallas_call(
        _argmax_body,
        grid=(steps,),
        in_specs=[
            pl.BlockSpec((_TOK_BLOCK, d), lambda i: (i, 0)),
            pl.BlockSpec((d, ncb), lambda i: (0, 0)),
        ],
        out_specs=[
            pl.BlockSpec((_TOK_BLOCK // _CHUNK, _CHUNK), lambda i: (i, 0)),
            pl.BlockSpec(memory_space=pltpu.SMEM),
        ],
        out_shape=[
            jax.ShapeDtypeStruct((n // _CHUNK, _CHUNK), jnp.int32),
            jax.ShapeDtypeStruct((1, 2), jnp.float32),
        ],
        scratch_shapes=[
            pltpu.SMEM((2,), jnp.float32),
        ],
    )(x, cbnt)


def _cbs_body(cb_ref, sums_ref, cbs_ref, loss_ref, *, n_tok):
    n = jnp.float32(n_tok)
    d = cb_ref.shape[1]
    loss = (1.0 + _COMMIT) * (2.0 * n - 2.0 * sums_ref[0, 0]) / (
        n * jnp.float32(d))
    loss_ref[0, 0] = loss
    scale = sums_ref[0, 1] / n
    cb = cb_ref[...]
    rn = jnp.sqrt(jnp.sum(cb * cb, axis=1, keepdims=True))
    cbs_ref[...] = cb * (scale / jnp.maximum(rn, _EPS))


def _run_phase2(cb, sums, n_tok):
    ncb, d = cb.shape
    return pl.pallas_call(
        functools.partial(_cbs_body, n_tok=n_tok),
        in_specs=[
            pl.BlockSpec((ncb, d), lambda: (0, 0)),
            pl.BlockSpec(memory_space=pltpu.SMEM),
        ],
        out_specs=[
            pl.BlockSpec((ncb, d), lambda: (0, 0)),
            pl.BlockSpec(memory_space=pltpu.SMEM),
        ],
        out_shape=[
            jax.ShapeDtypeStruct((ncb, d), jnp.float32),
            jax.ShapeDtypeStruct((1, 1), jnp.float32),
        ],
    )(cb, sums)


def _run_sc(cbs, idx2d, zeros, b, k, d, ncb):
    n_tok = b * k
    per_w = n_tok // _NW
    rows_half = per_w // 2
    nchunk = per_w // _CHUNK
    hist_cols = ncb // 2
    mesh = plsc.VectorSubcoreMesh(core_axis_name="c", subcore_axis_name="s")
    cp = pltpu.CompilerParams()
    if "needs_layout_passes" in pltpu.CompilerParams.__dataclass_fields__:
        cp = dataclasses.replace(cp, needs_layout_passes=False)
    if "use_tc_tiling_on_sc" in pltpu.CompilerParams.__dataclass_fields__:
        cp = dataclasses.replace(cp, use_tc_tiling_on_sc=False)

    @functools.partial(
        pl.kernel,
        mesh=mesh,
        compiler_params=cp,
        out_type=[
            jax.ShapeDtypeStruct((n_tok // 4, 128), jnp.float32),
            jax.ShapeDtypeStruct((2, 2, 16, hist_cols), jnp.float32),
        ],
        scratch_types=[
            pltpu.VMEM((nchunk, _CHUNK), jnp.int32),
            pltpu.VMEM((rows_half, d), jnp.float32),
            pltpu.VMEM((16, hist_cols), jnp.float32),
            pltpu.VMEM_SHARED((16, hist_cols), jnp.float32),
            pltpu.SemaphoreType.DMA,
        ],
    )
    def sc_kernel(cbs_hbm, idx_hbm, zeros_hbm, q_hbm, cnt_hbm,
                  idx_v, rows_v, hist_v, sh, sem):
        cid = lax.axis_index("c")
        sid = lax.axis_index("s")
        wid = sid * 2 + cid
        pltpu.sync_copy(idx_hbm.at[pl.ds(wid * nchunk, nchunk)], idx_v)
        ones = jnp.full((16,), 1.0, jnp.float32)
        lanes = lax.iota(jnp.int32, 16)
        half_n = nchunk // 2

        for half in range(2):
            copies = []
            for j in range(half_n):
                copies.append(pltpu.async_copy(
                    cbs_hbm.at[idx_v.at[half * half_n + j]],
                    rows_v.at[pl.ds(j * _CHUNK, _CHUNK)], sem))
            pltpu.sync_copy(zeros_hbm, hist_v)
            lo = jnp.int32(half * hist_cols)
            hi = jnp.int32((half + 1) * hist_cols)
            for j in range(nchunk):
                for kk in range(_CHUNK // 16):
                    vec = idx_v[j, pl.ds(kk * 16, 16)]
                    mask = (vec >= lo) & (vec < hi)
                    plsc.addupdate_scatter(hist_v, [lanes, vec - lo], ones,
                                           mask=mask)
            @pl.when(sid == 0)
            def _zero_shared():
                pltpu.sync_copy(zeros_hbm, sh)
            plsc.subcore_barrier()
            pltpu.sync_copy(hist_v, sh.at[lanes], add=True)
            plsc.subcore_barrier()

            @pl.when(sid == 0)
            def _spill_counts():
                pltpu.sync_copy(sh, cnt_hbm.at[half, cid])

            for cp_ in copies:
                cp_.wait()
            bidx = wid * (per_w // k) + half
            qlines = k // 4
            for j in range(4):
                pltpu.sync_copy(
                    rows_v.at[pl.ds(j * qlines, qlines)],
                    q_hbm.at[pl.ds(bidx * qlines, qlines),
                             pl.ds(j * d, d)])

    return sc_kernel(cbs, idx2d, zeros)


def _qt_body(q_ref, out_ref):
    nb, d, k = out_ref.shape
    lines = k // 4
    for bb in range(nb):
        for j in range(128 // d):
            sub = q_ref[pl.ds(bb * lines, lines), pl.ds(j * d, d)]
            out_ref[bb, :, pl.ds(j * lines, lines)] = sub.T


def _run_qt(q4, b, k, d):
    nb = 8
    lines = q4.shape[0] // b
    return pl.pallas_call(
        _qt_body,
        grid=(b // nb,),
        in_specs=[pl.BlockSpec((nb * lines, 128), lambda i: (i, 0))],
        out_specs=pl.BlockSpec((nb, d, k), lambda i: (i, 0, 0)),
        out_shape=jax.ShapeDtypeStruct((b, d, k), jnp.float32),
    )(q4)


def _perp_body(cnt_ref, perp_ref, *, n_tok):
    c = cnt_ref[...]
    rows = c.shape[0] // 2
    c0 = jnp.sum(c[:rows], axis=0)
    c1 = jnp.sum(c[rows:], axis=0)
    inv = jnp.float32(1.0 / n_tok)
    p0 = c0 * inv
    p1 = c1 * inv
    ent = jnp.sum(p0 * jnp.log(p0 + 1e-10)) + jnp.sum(p1 * jnp.log(p1 + 1e-10))
    perp_ref[...] = jnp.reshape(jnp.exp(-ent), (1, 1))


def _run_phase3(cnt2d, n_tok):
    return pl.pallas_call(
        functools.partial(_perp_body, n_tok=n_tok),
        out_shape=jax.ShapeDtypeStruct((1, 1), jnp.float32),
    )(cnt2d)


def kernel(inputs, codebook):
    b, k, d = inputs.shape
    n_tok = b * k
    ncb = codebook.shape[0]
    x = inputs.reshape(n_tok, d).astype(jnp.float32)
    cb = codebook.astype(jnp.float32)

    cbnt = _run_prep(cb.T)
    idx2d, sums = _run_phase1(x, cbnt)
    cbs, loss11 = _run_phase2(cb, sums, n_tok)
    zeros = jnp.zeros((16, ncb // 2), jnp.float32)
    q, cnt = _run_sc(cbs, idx2d, zeros, b, k, d, ncb)
    perp11 = _run_phase3(cnt.reshape(2 * 2 * 16, ncb // 2), n_tok)
    qt = _run_qt(q, b, k, d)

    return (jnp.swapaxes(qt, 1, 2), loss11[0, 0], perp11[0, 0])
